# baseline (speedup 1.0000x reference)
import functools

import jax
import jax.numpy as jnp
from jax import lax
from jax.experimental import pallas as pl
from jax.experimental.pallas import tpu as pltpu
from jax.experimental.pallas import tpu_sc as plsc

F32 = jnp.float32

N_PAD = 50176
E_PAD = 819200
BA = 512
BB = 256
BC = 128
BN = 512
BE = 2048

_MESH = plsc.VectorSubcoreMesh(core_axis_name="c", subcore_axis_name="s")


def _agg_pipeline(tbl_h, src_h, dst3_h, acc, srcv, dstv, rows,
                  gsem, ssem, ebase, nb, ba, gsz):
    ngroups = nb // gsz
    npairs = gsz // 2
    rpb = ba // 128

    def fire_gather(j, s):
        pltpu.async_copy(tbl_h.at[srcv.at[pl.ds(j * ba, ba)]],
                         rows[s], gsem[s])

    def wait_gather(s):
        pltpu.make_async_copy(tbl_h.at[srcv.at[pl.ds(0, ba)]],
                              rows[s], gsem[s]).wait()

    def fire_scatter(j, s):
        return [pltpu.async_copy(rows[s].at[pl.ds(k * 128, 128)],
                                 acc.at[dstv.at[j * rpb + k]],
                                 ssem[s], add=True)
                for k in range(rpb)]

    def group(g, carry):
        base = ebase + g * gsz * ba
        pltpu.sync_copy(src_h.at[pl.ds(base, gsz * ba)], srcv)
        row = pl.multiple_of(ebase // 128 + g * (gsz * rpb), 2)
        pltpu.sync_copy(dst3_h.at[pl.ds(row, gsz * rpb)], dstv)
        fire_gather(0, 0)
        fire_gather(1, 1)
        for p in range(npairs):
            b0 = 2 * p
            wait_gather(0)
            sc0 = fire_scatter(b0, 0)
            wait_gather(1)
            sc1 = fire_scatter(b0 + 1, 1)
            for d in sc0:
                d.wait()
            if p < npairs - 1:
                fire_gather(b0 + 2, 0)
            for d in sc1:
                d.wait()
            if p < npairs - 1:
                fire_gather(b0 + 3, 1)
        return carry

    lax.fori_loop(0, ngroups, group, 0)


def _sck_l1(xp, src, dst3, z16):
    stripe = N_PAD // 16

    @functools.partial(
        pl.kernel,
        out_type=jax.ShapeDtypeStruct((2, N_PAD, 128), F32),
        mesh=_MESH,
        compiler_params=pltpu.CompilerParams(use_tc_tiling_on_sc=False),
        scratch_types=[
            pltpu.VMEM_SHARED((N_PAD, 16), F32),
            pltpu.VMEM((10 * BA,), jnp.int32),
            pltpu.VMEM((10 * BA // 128, 128), jnp.int32),
            [pltpu.VMEM((BA, 16), F32)] * 2,
            [pltpu.SemaphoreType.DMA] * 2,
            [pltpu.SemaphoreType.DMA] * 2,
        ],
    )
    def k(xp_h, src_h, dst3_h, z_h, out_h, acc, srcv, dstv, rows,
          gsem, ssem):
        cid = lax.axis_index("c")
        sid = lax.axis_index("s")
        pltpu.sync_copy(z_h, acc.at[pl.ds(sid * stripe, stripe)])
        plsc.subcore_barrier()
        ebase = cid * (E_PAD // 2) + sid * (E_PAD // 32)
        nb = E_PAD // 32 // BA
        _agg_pipeline(xp_h, src_h, dst3_h, acc, srcv, dstv, rows,
                      gsem, ssem, ebase, nb, BA, 10)
        plsc.subcore_barrier()
        pltpu.sync_copy(acc.at[pl.ds(sid * stripe, stripe)],
                        out_h.at[cid, pl.ds(sid * stripe, stripe),
                                 pl.ds(0, 16)])

    return k(xp, src, dst3, z16)


def _sck_agg(t0, t1, t2, t3, src, dst3, z32):
    stripe = N_PAD // 16

    @functools.partial(
        pl.kernel,
        out_type=jax.ShapeDtypeStruct((N_PAD, 128), F32),
        mesh=_MESH,
        compiler_params=pltpu.CompilerParams(use_tc_tiling_on_sc=False),
        scratch_types=[
            pltpu.VMEM_SHARED((N_PAD, 32), F32),
            pltpu.VMEM((10 * BB,), jnp.int32),
            pltpu.VMEM((10 * BB // 128, 128), jnp.int32),
            [pltpu.VMEM((BB, 32), F32)] * 2,
            [pltpu.SemaphoreType.DMA] * 2,
            [pltpu.SemaphoreType.DMA] * 2,
        ],
    )
    def k(t0_h, t1_h, t2_h, t3_h, src_h, dst3_h, z_h, out_h,
          acc, srcv, dstv, rows, gsem, ssem):
        cid = lax.axis_index("c")
        sid = lax.axis_index("s")
        ebase = sid * (E_PAD // 16)
        nb = E_PAD // 16 // BB

        def do_chunk(tbl_h, c):
            pltpu.sync_copy(z_h, acc.at[pl.ds(sid * stripe, stripe)])
            plsc.subcore_barrier()
            _agg_pipeline(tbl_h, src_h, dst3_h, acc, srcv, dstv, rows,
                          gsem, ssem, ebase, nb, BB, 10)
            plsc.subcore_barrier()
            pltpu.sync_copy(acc.at[pl.ds(sid * stripe, stripe)],
                            out_h.at[pl.ds(sid * stripe, stripe),
                                     pl.ds(32 * c, 32)])

        @pl.when(cid == 0)
        def _():
            do_chunk(t0_h, 0)
            do_chunk(t1_h, 1)

        @pl.when(cid == 1)
        def _():
            do_chunk(t2_h, 2)
            do_chunk(t3_h, 3)

    return k(t0, t1, t2, t3, src, dst3, z32)


def _sck_edge_gather(a, b, src, dst):
    GSZ = 10

    @functools.partial(
        pl.kernel,
        out_type=jax.ShapeDtypeStruct((E_PAD, 128), F32),
        mesh=_MESH,
        compiler_params=pltpu.CompilerParams(use_tc_tiling_on_sc=True),
        scratch_types=[
            pltpu.VMEM((GSZ * BC,), jnp.int32),
            pltpu.VMEM((GSZ * BC,), jnp.int32),
            [pltpu.VMEM((BC, 128), F32)] * 2,
            [pltpu.VMEM((BC, 128), F32)] * 2,
            [pltpu.SemaphoreType.DMA] * 2,
            [pltpu.SemaphoreType.DMA] * 2,
            [pltpu.SemaphoreType.DMA] * 2,
        ],
    )
    def k(a_h, b_h, src_h, dst_h, g_h,
          sidx, didx, bufa, bufb, sema, semb, semw):
        cid = lax.axis_index("c")
        sid = lax.axis_index("s")
        ebase = cid * (E_PAD // 2) + sid * (E_PAD // 32)
        nb = E_PAD // 32 // BC
        ngroups = nb // GSZ
        npairs = GSZ // 2

        def fire_gathers(j, s):
            pltpu.async_copy(a_h.at[sidx.at[pl.ds(j * BC, BC)]],
                             bufa[s], sema[s])
            pltpu.async_copy(b_h.at[didx.at[pl.ds(j * BC, BC)]],
                             bufb[s], semb[s])

        def wait_gathers(s):
            pltpu.make_async_copy(a_h.at[sidx.at[pl.ds(0, BC)]],
                                  bufa[s], sema[s]).wait()
            pltpu.make_async_copy(b_h.at[didx.at[pl.ds(0, BC)]],
                                  bufb[s], semb[s]).wait()

        def add_rows(s):
            def addrow(r, carry):
                for j in range(8):
                    sl = pl.ds(j * 16, 16)
                    bufa[s][r, sl] = bufa[s][r, sl] + bufb[s][r, sl]
                return carry

            lax.fori_loop(0, BC, addrow, 0)

        def group(g, carry):
            base = ebase + g * GSZ * BC
            pltpu.sync_copy(src_h.at[pl.ds(base, GSZ * BC)], sidx)
            pltpu.sync_copy(dst_h.at[pl.ds(base, GSZ * BC)], didx)
            fire_gathers(0, 0)
            fire_gathers(1, 1)
            for p in range(npairs):
                b0 = 2 * p
                wait_gathers(0)
                add_rows(0)
                w0 = pltpu.async_copy(
                    bufa[0], g_h.at[pl.ds(base + b0 * BC, BC)], semw[0])
                wait_gathers(1)
                add_rows(1)
                w1 = pltpu.async_copy(
                    bufa[1], g_h.at[pl.ds(base + (b0 + 1) * BC, BC)],
                    semw[1])
                w0.wait()
                if p < npairs - 1:
                    fire_gathers(b0 + 2, 0)
                w1.wait()
                if p < npairs - 1:
                    fire_gathers(b0 + 3, 1)
            return carry

        lax.fori_loop(0, ngroups, group, 0)

    return k(a, b, src, dst)


def _tck1(aggx, xp, w1lp, w1rp, b1r):
    grid = (N_PAD // BN,)

    def body(aggx_r, xp_r, wl_r, wr_r, b_r, h_o, rcp_r):
        s = aggx_r[0, :, :16] + aggx_r[1, :, :16]
        r = 1.0 / jnp.maximum(s[:, 6:7], 1.0)
        y = jnp.dot(s * r, wl_r[...], preferred_element_type=F32)
        y += jnp.dot(xp_r[...], wr_r[...], preferred_element_type=F32)
        h_o[...] = jnp.maximum(y + b_r[...], 0.0)
        rcp_r[...] = jnp.broadcast_to(r, (BN, 16))

    return pl.pallas_call(
        body,
        grid=grid,
        in_specs=[
            pl.BlockSpec((2, BN, 128), lambda i: (0, i, 0)),
            pl.BlockSpec((BN, 16), lambda i: (i, 0)),
            pl.BlockSpec((16, 128), lambda i: (0, 0)),
            pl.BlockSpec((16, 128), lambda i: (0, 0)),
            pl.BlockSpec((1, 128), lambda i: (0, 0)),
        ],
        out_specs=[pl.BlockSpec((BN, 128), lambda i: (i, 0)),
                   pl.BlockSpec((BN, 16), lambda i: (i, 0))],
        out_shape=(jax.ShapeDtypeStruct((N_PAD, 128), F32),
                   jax.ShapeDtypeStruct((N_PAD, 16), F32)),
    )(aggx, xp, w1lp, w1rp, b1r)


def _tck_sage(agg, h, rcp, wl, wr, br):
    grid = (N_PAD // BN,)

    def body(agg_r, h_r, rcp_r, wl_r, wr_r, b_r, o_r):
        mean = agg_r[...] * rcp_r[:, 0:1]
        y = jnp.dot(mean, wl_r[...], preferred_element_type=F32)
        y += jnp.dot(h_r[...], wr_r[...], preferred_element_type=F32)
        o_r[...] = jnp.maximum(y + b_r[...], 0.0)

    return pl.pallas_call(
        body,
        grid=grid,
        in_specs=[
            pl.BlockSpec((BN, 128), lambda i: (i, 0)),
            pl.BlockSpec((BN, 128), lambda i: (i, 0)),
            pl.BlockSpec((BN, 16), lambda i: (i, 0)),
            pl.BlockSpec((128, 128), lambda i: (0, 0)),
            pl.BlockSpec((128, 128), lambda i: (0, 0)),
            pl.BlockSpec((1, 128), lambda i: (0, 0)),
        ],
        out_specs=pl.BlockSpec((BN, 128), lambda i: (i, 0)),
        out_shape=jax.ShapeDtypeStruct((N_PAD, 128), F32),
    )(agg, h, rcp, wl, wr, br)


def _tck3(agg, h, rcp, wl, wr, br, we1l, we1r):
    grid = (N_PAD // BN,)

    def body(agg_r, h_r, rcp_r, wl_r, wr_r, b_r,
             wel_r, wer_r, a_o, b_o):
        h = h_r[...]
        mean = agg_r[...] * rcp_r[:, 0:1]
        y = jnp.dot(mean, wl_r[...], preferred_element_type=F32)
        y += jnp.dot(h, wr_r[...], preferred_element_type=F32)
        y = jnp.maximum(y + b_r[...], 0.0) + h
        a_o[...] = jnp.dot(y, wel_r[...], preferred_element_type=F32)
        b_o[...] = jnp.dot(y, wer_r[...], preferred_element_type=F32)

    full_spec = pl.BlockSpec((BN, 128), lambda i: (i, 0))
    w_spec = pl.BlockSpec((128, 128), lambda i: (0, 0))
    return pl.pallas_call(
        body,
        grid=grid,
        in_specs=[
            pl.BlockSpec((BN, 128), lambda i: (i, 0)),
            full_spec,
            pl.BlockSpec((BN, 16), lambda i: (i, 0)),
            w_spec, w_spec,
            pl.BlockSpec((1, 128), lambda i: (0, 0)),
            w_spec, w_spec,
        ],
        out_specs=[full_spec, full_spec],
        out_shape=(jax.ShapeDtypeStruct((N_PAD, 128), F32),
                   jax.ShapeDtypeStruct((N_PAD, 128), F32)),
    )(agg, h, rcp, wl, wr, br, we1l, we1r)


def _tck4(g, ea, we1e, be1r, we2, be2r, w3r, be3, E):
    grid = (pl.cdiv(E, BE),)

    def body(g_r, ea_r, we1_r, b1_r, we2_r, b2_r, w3_r, b3_s, o_r):
        e1 = g_r[...]
        e1 += lax.dot_general(ea_r[...], we1_r[...],
                              (((0,), (0,)), ((), ())),
                              preferred_element_type=F32)
        e1 = jnp.maximum(e1 + b1_r[...], 0.0)
        e2 = jnp.dot(e1, we2_r[...], preferred_element_type=F32)
        e2 = jnp.maximum(e2 + b2_r[...], 0.0)
        o_r[...] = lax.dot_general(w3_r[...], e2, (((0,), (1,)), ((), ())),
                                   preferred_element_type=F32) + b3_s[0]

    return pl.pallas_call(
        body,
        grid=grid,
        in_specs=[
            pl.BlockSpec((BE, 128), lambda i: (i, 0)),
            pl.BlockSpec((11, BE), lambda i: (0, i)),
            pl.BlockSpec((11, 128), lambda i: (0, 0)),
            pl.BlockSpec((1, 128), lambda i: (0, 0)),
            pl.BlockSpec((128, 64), lambda i: (0, 0)),
            pl.BlockSpec((1, 64), lambda i: (0, 0)),
            pl.BlockSpec((64, 1), lambda i: (0, 0)),
            pl.BlockSpec(memory_space=pltpu.SMEM),
        ],
        out_specs=pl.BlockSpec((1, BE), lambda i: (0, i)),
        out_shape=jax.ShapeDtypeStruct((1, E), F32),
    )(g, ea, we1e, be1r, we2, be2r, w3r, be3)


def kernel(x, edge_index, edge_attr, W1l, W1r, b1, W2l, W2r, b2,
           W3l, W3r, b3, We1, be1, We2, be2, We3, be3):
    N = x.shape[0]
    E = edge_index.shape[1]

    xp = jnp.zeros((N_PAD, 16), F32)
    xp = xp.at[:N, :6].set(x)
    xp = xp.at[:, 6].set(1.0)

    src = edge_index[0].astype(jnp.int32)
    dst = edge_index[1].astype(jnp.int32)
    npad = E_PAD - E
    ar = lax.iota(jnp.int32, npad)
    src_p = jnp.concatenate([src, ar % N])
    dst_p = jnp.concatenate([dst, N + ar % (N_PAD - N)])
    dst3 = dst_p.reshape(E_PAD // 128, 128)

    z16 = jnp.zeros((N_PAD // 16, 16), F32)
    z32 = jnp.zeros((N_PAD // 16, 32), F32)

    pad_w = lambda w: jnp.zeros((16, 128), F32).at[:w.shape[0]].set(w)
    w1lp, w1rp = pad_w(W1l), pad_w(W1r)
    b1r, b2r, b3r = b1.reshape(1, 128), b2.reshape(1, 128), b3.reshape(1, 128)
    we1l, we1r, we1e = We1[:128], We1[128:256], We1[256:]
    be1r, be2r = be1.reshape(1, 128), be2.reshape(1, 64)
    w3r = We3

    chunks = lambda t: tuple(t[:, 32 * c:32 * (c + 1)] for c in range(4))

    aggx = _sck_l1(xp, src_p, dst3, z16)
    h, rcp = _tck1(aggx, xp, w1lp, w1rp, b1r)

    agg2 = _sck_agg(*chunks(h), src_p, dst3, z32)
    h2 = _tck_sage(agg2, h, rcp, W2l, W2r, b2r)

    agg3 = _sck_agg(*chunks(h2), src_p, dst3, z32)
    a, b = _tck3(agg3, h2, rcp, W3l, W3r, b3r, we1l, we1r)

    g = _sck_edge_gather(a, b, src_p, dst_p)
    return _tck4(g, edge_attr.T, we1e, be1r, We2, be2r, w3r, be3, E)[0]


# speedup vs baseline: 1.0616x; 1.0616x over previous
import functools

import jax
import jax.numpy as jnp
from jax import lax
from jax.experimental import pallas as pl
from jax.experimental.pallas import tpu as pltpu
from jax.experimental.pallas import tpu_sc as plsc

F32 = jnp.float32

N_PAD = 50176
E_PAD = 819200
BA = 512
BB = 256
BC = 128
BN = 512
BE = 4096

_MESH = plsc.VectorSubcoreMesh(core_axis_name="c", subcore_axis_name="s")


def _agg_pipeline(tbl_h, src_h, dst3_h, acc, srcv, dstv, rows,
                  gsem, ssem, ebase, nb, ba, gsz):
    ngroups = nb // gsz
    npairs = gsz // 2
    rpb = ba // 128

    def fire_gather(j, s):
        pltpu.async_copy(tbl_h.at[srcv.at[pl.ds(j * ba, ba)]],
                         rows[s], gsem[s])

    def wait_gather(s):
        pltpu.make_async_copy(tbl_h.at[srcv.at[pl.ds(0, ba)]],
                              rows[s], gsem[s]).wait()

    def fire_scatter(j, s):
        return [pltpu.async_copy(rows[s].at[pl.ds(k * 128, 128)],
                                 acc.at[dstv.at[j * rpb + k]],
                                 ssem[s], add=True)
                for k in range(rpb)]

    def group(g, carry):
        base = ebase + g * gsz * ba
        pltpu.sync_copy(src_h.at[pl.ds(base, gsz * ba)], srcv)
        row = pl.multiple_of(ebase // 128 + g * (gsz * rpb), 2)
        pltpu.sync_copy(dst3_h.at[pl.ds(row, gsz * rpb)], dstv)
        fire_gather(0, 0)
        fire_gather(1, 1)
        for p in range(npairs):
            b0 = 2 * p
            wait_gather(0)
            sc0 = fire_scatter(b0, 0)
            wait_gather(1)
            sc1 = fire_scatter(b0 + 1, 1)
            for d in sc0:
                d.wait()
            if p < npairs - 1:
                fire_gather(b0 + 2, 0)
            for d in sc1:
                d.wait()
            if p < npairs - 1:
                fire_gather(b0 + 3, 1)
        return carry

    lax.fori_loop(0, ngroups, group, 0)


def _sck_l1(xp, src, dst3, z16):
    stripe = N_PAD // 16

    @functools.partial(
        pl.kernel,
        out_type=jax.ShapeDtypeStruct((2, N_PAD, 128), F32),
        mesh=_MESH,
        compiler_params=pltpu.CompilerParams(use_tc_tiling_on_sc=False),
        scratch_types=[
            pltpu.VMEM_SHARED((N_PAD, 16), F32),
            pltpu.VMEM((10 * BA,), jnp.int32),
            pltpu.VMEM((10 * BA // 128, 128), jnp.int32),
            [pltpu.VMEM((BA, 16), F32)] * 2,
            [pltpu.SemaphoreType.DMA] * 2,
            [pltpu.SemaphoreType.DMA] * 2,
        ],
    )
    def k(xp_h, src_h, dst3_h, z_h, out_h, acc, srcv, dstv, rows,
          gsem, ssem):
        cid = lax.axis_index("c")
        sid = lax.axis_index("s")
        pltpu.sync_copy(z_h, acc.at[pl.ds(sid * stripe, stripe)])
        plsc.subcore_barrier()
        ebase = cid * (E_PAD // 2) + sid * (E_PAD // 32)
        nb = E_PAD // 32 // BA
        _agg_pipeline(xp_h, src_h, dst3_h, acc, srcv, dstv, rows,
                      gsem, ssem, ebase, nb, BA, 10)
        plsc.subcore_barrier()
        pltpu.sync_copy(acc.at[pl.ds(sid * stripe, stripe)],
                        out_h.at[cid, pl.ds(sid * stripe, stripe),
                                 pl.ds(0, 16)])

    return k(xp, src, dst3, z16)


def _sck_agg(t0, t1, t2, t3, src, dst3, z32):
    stripe = N_PAD // 16

    @functools.partial(
        pl.kernel,
        out_type=jax.ShapeDtypeStruct((N_PAD, 128), F32),
        mesh=_MESH,
        compiler_params=pltpu.CompilerParams(use_tc_tiling_on_sc=False),
        scratch_types=[
            pltpu.VMEM_SHARED((N_PAD, 32), F32),
            pltpu.VMEM((10 * BB,), jnp.int32),
            pltpu.VMEM((10 * BB // 128, 128), jnp.int32),
            [pltpu.VMEM((BB, 32), F32)] * 2,
            [pltpu.SemaphoreType.DMA] * 2,
            [pltpu.SemaphoreType.DMA] * 2,
        ],
    )
    def k(t0_h, t1_h, t2_h, t3_h, src_h, dst3_h, z_h, out_h,
          acc, srcv, dstv, rows, gsem, ssem):
        cid = lax.axis_index("c")
        sid = lax.axis_index("s")
        ebase = sid * (E_PAD // 16)
        nb = E_PAD // 16 // BB

        def do_chunk(tbl_h, c):
            pltpu.sync_copy(z_h, acc.at[pl.ds(sid * stripe, stripe)])
            plsc.subcore_barrier()
            _agg_pipeline(tbl_h, src_h, dst3_h, acc, srcv, dstv, rows,
                          gsem, ssem, ebase, nb, BB, 10)
            plsc.subcore_barrier()
            pltpu.sync_copy(acc.at[pl.ds(sid * stripe, stripe)],
                            out_h.at[pl.ds(sid * stripe, stripe),
                                     pl.ds(32 * c, 32)])

        @pl.when(cid == 0)
        def _():
            do_chunk(t0_h, 0)
            do_chunk(t1_h, 1)

        @pl.when(cid == 1)
        def _():
            do_chunk(t2_h, 2)
            do_chunk(t3_h, 3)

    return k(t0, t1, t2, t3, src, dst3, z32)


def _sck_edge_gather(a, b, src, dst):
    GSZ = 10

    @functools.partial(
        pl.kernel,
        out_type=jax.ShapeDtypeStruct((E_PAD, 128), F32),
        mesh=_MESH,
        compiler_params=pltpu.CompilerParams(use_tc_tiling_on_sc=True),
        scratch_types=[
            pltpu.VMEM((GSZ * BC,), jnp.int32),
            pltpu.VMEM((GSZ * BC,), jnp.int32),
            [pltpu.VMEM((BC, 128), F32)] * 2,
            [pltpu.VMEM((BC, 128), F32)] * 2,
            [pltpu.SemaphoreType.DMA] * 2,
            [pltpu.SemaphoreType.DMA] * 2,
            [pltpu.SemaphoreType.DMA] * 2,
        ],
    )
    def k(a_h, b_h, src_h, dst_h, g_h,
          sidx, didx, bufa, bufb, sema, semb, semw):
        cid = lax.axis_index("c")
        sid = lax.axis_index("s")
        ebase = cid * (E_PAD // 2) + sid * (E_PAD // 32)
        nb = E_PAD // 32 // BC
        ngroups = nb // GSZ
        npairs = GSZ // 2

        def fire_gathers(j, s):
            pltpu.async_copy(a_h.at[sidx.at[pl.ds(j * BC, BC)]],
                             bufa[s], sema[s])
            pltpu.async_copy(b_h.at[didx.at[pl.ds(j * BC, BC)]],
                             bufb[s], semb[s])

        def wait_gathers(s):
            pltpu.make_async_copy(a_h.at[sidx.at[pl.ds(0, BC)]],
                                  bufa[s], sema[s]).wait()
            pltpu.make_async_copy(b_h.at[didx.at[pl.ds(0, BC)]],
                                  bufb[s], semb[s]).wait()

        def add_rows(s):
            def addrow(r, carry):
                for j in range(8):
                    sl = pl.ds(j * 16, 16)
                    bufa[s][r, sl] = bufa[s][r, sl] + bufb[s][r, sl]
                return carry

            lax.fori_loop(0, BC, addrow, 0)

        def group(g, carry):
            base = ebase + g * GSZ * BC
            pltpu.sync_copy(src_h.at[pl.ds(base, GSZ * BC)], sidx)
            pltpu.sync_copy(dst_h.at[pl.ds(base, GSZ * BC)], didx)
            fire_gathers(0, 0)
            fire_gathers(1, 1)
            for p in range(npairs):
                b0 = 2 * p
                wait_gathers(0)
                add_rows(0)
                w0 = pltpu.async_copy(
                    bufa[0], g_h.at[pl.ds(base + b0 * BC, BC)], semw[0])
                wait_gathers(1)
                add_rows(1)
                w1 = pltpu.async_copy(
                    bufa[1], g_h.at[pl.ds(base + (b0 + 1) * BC, BC)],
                    semw[1])
                w0.wait()
                if p < npairs - 1:
                    fire_gathers(b0 + 2, 0)
                w1.wait()
                if p < npairs - 1:
                    fire_gathers(b0 + 3, 1)
            return carry

        lax.fori_loop(0, ngroups, group, 0)

    return k(a, b, src, dst)


def _tck1(aggx, xp, w1lp, w1rp, b1r):
    grid = (N_PAD // BN,)

    def body(aggx_r, xp_r, wl_r, wr_r, b_r, h0, h1, h2, h3, rcp_r):
        s = aggx_r[0, :, :16] + aggx_r[1, :, :16]
        r = 1.0 / jnp.maximum(s[:, 6:7], 1.0)
        y = jnp.dot(s * r, wl_r[...], preferred_element_type=F32)
        y += jnp.dot(xp_r[...], wr_r[...], preferred_element_type=F32)
        y = jnp.maximum(y + b_r[...], 0.0)
        for c, h in enumerate((h0, h1, h2, h3)):
            h[...] = y[:, 32 * c:32 * (c + 1)]
        rcp_r[...] = jnp.broadcast_to(r, (BN, 16))

    out_shape = tuple(jax.ShapeDtypeStruct((N_PAD, 32), F32)
                      for _ in range(4))
    out_shape += (jax.ShapeDtypeStruct((N_PAD, 16), F32),)
    chunk_spec = pl.BlockSpec((BN, 32), lambda i: (i, 0))
    return pl.pallas_call(
        body,
        grid=grid,
        in_specs=[
            pl.BlockSpec((2, BN, 128), lambda i: (0, i, 0)),
            pl.BlockSpec((BN, 16), lambda i: (i, 0)),
            pl.BlockSpec((16, 128), lambda i: (0, 0)),
            pl.BlockSpec((16, 128), lambda i: (0, 0)),
            pl.BlockSpec((1, 128), lambda i: (0, 0)),
        ],
        out_specs=[chunk_spec] * 4 + [pl.BlockSpec((BN, 16), lambda i: (i, 0))],
        out_shape=out_shape,
    )(aggx, xp, w1lp, w1rp, b1r)


def _tck_sage(agg, h0, h1, h2, h3, rcp, wl, wr, br):
    grid = (N_PAD // BN,)

    def body(agg_r, h0_r, h1_r, h2_r, h3_r, rcp_r, wl_r, wr_r, b_r,
             o0, o1, o2, o3):
        aggc = agg_r[...]
        h = jnp.concatenate([h0_r[...], h1_r[...], h2_r[...], h3_r[...]],
                            axis=-1)
        mean = aggc * rcp_r[:, 0:1]
        y = jnp.dot(mean, wl_r[...], preferred_element_type=F32)
        y += jnp.dot(h, wr_r[...], preferred_element_type=F32)
        y = jnp.maximum(y + b_r[...], 0.0)
        for c, o in enumerate((o0, o1, o2, o3)):
            o[...] = y[:, 32 * c:32 * (c + 1)]

    chunk_spec = pl.BlockSpec((BN, 32), lambda i: (i, 0))
    return pl.pallas_call(
        body,
        grid=grid,
        in_specs=[
            pl.BlockSpec((BN, 128), lambda i: (i, 0)),
            chunk_spec, chunk_spec, chunk_spec, chunk_spec,
            pl.BlockSpec((BN, 16), lambda i: (i, 0)),
            pl.BlockSpec((128, 128), lambda i: (0, 0)),
            pl.BlockSpec((128, 128), lambda i: (0, 0)),
            pl.BlockSpec((1, 128), lambda i: (0, 0)),
        ],
        out_specs=[chunk_spec] * 4,
        out_shape=tuple(jax.ShapeDtypeStruct((N_PAD, 32), F32)
                        for _ in range(4)),
    )(agg, h0, h1, h2, h3, rcp, wl, wr, br)


def _tck3(agg, h0, h1, h2, h3, rcp, wl, wr, br, we1l, we1r):
    grid = (N_PAD // BN,)

    def body(agg_r, h0_r, h1_r, h2_r, h3_r, rcp_r, wl_r, wr_r, b_r,
             wel_r, wer_r, a_o, b_o):
        aggc = agg_r[...]
        h = jnp.concatenate([h0_r[...], h1_r[...], h2_r[...], h3_r[...]],
                            axis=-1)
        mean = aggc * rcp_r[:, 0:1]
        y = jnp.dot(mean, wl_r[...], preferred_element_type=F32)
        y += jnp.dot(h, wr_r[...], preferred_element_type=F32)
        y = jnp.maximum(y + b_r[...], 0.0) + h
        a_o[...] = jnp.dot(y, wel_r[...], preferred_element_type=F32)
        b_o[...] = jnp.dot(y, wer_r[...], preferred_element_type=F32)

    chunk_spec = pl.BlockSpec((BN, 32), lambda i: (i, 0))
    full_spec = pl.BlockSpec((BN, 128), lambda i: (i, 0))
    w_spec = pl.BlockSpec((128, 128), lambda i: (0, 0))
    return pl.pallas_call(
        body,
        grid=grid,
        in_specs=[
            pl.BlockSpec((BN, 128), lambda i: (i, 0)),
            chunk_spec, chunk_spec, chunk_spec, chunk_spec,
            pl.BlockSpec((BN, 16), lambda i: (i, 0)),
            w_spec, w_spec,
            pl.BlockSpec((1, 128), lambda i: (0, 0)),
            w_spec, w_spec,
        ],
        out_specs=[full_spec, full_spec],
        out_shape=(jax.ShapeDtypeStruct((N_PAD, 128), F32),
                   jax.ShapeDtypeStruct((N_PAD, 128), F32)),
    )(agg, h0, h1, h2, h3, rcp, wl, wr, br, we1l, we1r)


def _tck4(g, ea, we1e, be1r, we2, be2r, w3r, be3, E):
    grid = (pl.cdiv(E, BE),)

    def body(g_r, ea_r, we1_r, b1_r, we2_r, b2_r, w3_r, b3_s, o_r):
        e1 = g_r[...]
        e1 += lax.dot_general(ea_r[...], we1_r[...],
                              (((0,), (0,)), ((), ())),
                              preferred_element_type=F32)
        e1 = jnp.maximum(e1 + b1_r[...], 0.0)
        e2 = jnp.dot(e1, we2_r[...], preferred_element_type=F32)
        e2 = jnp.maximum(e2 + b2_r[...], 0.0)
        o_r[...] = lax.dot_general(w3_r[...], e2, (((0,), (1,)), ((), ())),
                                   preferred_element_type=F32) + b3_s[0]

    return pl.pallas_call(
        body,
        grid=grid,
        in_specs=[
            pl.BlockSpec((BE, 128), lambda i: (i, 0)),
            pl.BlockSpec((11, BE), lambda i: (0, i)),
            pl.BlockSpec((11, 128), lambda i: (0, 0)),
            pl.BlockSpec((1, 128), lambda i: (0, 0)),
            pl.BlockSpec((128, 64), lambda i: (0, 0)),
            pl.BlockSpec((1, 64), lambda i: (0, 0)),
            pl.BlockSpec((64, 1), lambda i: (0, 0)),
            pl.BlockSpec(memory_space=pltpu.SMEM),
        ],
        out_specs=pl.BlockSpec((1, BE), lambda i: (0, i)),
        out_shape=jax.ShapeDtypeStruct((1, E), F32),
    )(g, ea, we1e, be1r, we2, be2r, w3r, be3)


def kernel(x, edge_index, edge_attr, W1l, W1r, b1, W2l, W2r, b2,
           W3l, W3r, b3, We1, be1, We2, be2, We3, be3):
    N = x.shape[0]
    E = edge_index.shape[1]

    xp = jnp.zeros((N_PAD, 16), F32)
    xp = xp.at[:N, :6].set(x)
    xp = xp.at[:, 6].set(1.0)

    src = edge_index[0].astype(jnp.int32)
    dst = edge_index[1].astype(jnp.int32)
    npad = E_PAD - E
    ar = lax.iota(jnp.int32, npad)
    src_p = jnp.concatenate([src, ar % N])
    dst_p = jnp.concatenate([dst, N + ar % (N_PAD - N)])
    dst3 = dst_p.reshape(E_PAD // 128, 128)

    z16 = jnp.zeros((N_PAD // 16, 16), F32)
    z32 = jnp.zeros((N_PAD // 16, 32), F32)

    pad_w = lambda w: jnp.zeros((16, 128), F32).at[:w.shape[0]].set(w)
    w1lp, w1rp = pad_w(W1l), pad_w(W1r)
    b1r, b2r, b3r = b1.reshape(1, 128), b2.reshape(1, 128), b3.reshape(1, 128)
    we1l, we1r, we1e = We1[:128], We1[128:256], We1[256:]
    be1r, be2r = be1.reshape(1, 128), be2.reshape(1, 64)
    w3r = We3

    aggx = _sck_l1(xp, src_p, dst3, z16)
    h = _tck1(aggx, xp, w1lp, w1rp, b1r)
    hc, rcp = h[:4], h[4]

    agg2 = _sck_agg(*hc, src_p, dst3, z32)
    h2c = _tck_sage(agg2, *hc, rcp, W2l, W2r, b2r)

    agg3 = _sck_agg(*h2c, src_p, dst3, z32)
    a, b = _tck3(agg3, *h2c, rcp, W3l, W3r, b3r, we1l, we1r)

    g = _sck_edge_gather(a, b, src_p, dst_p)
    return _tck4(g, edge_attr.T, we1e, be1r, We2, be2r, w3r, be3, E)[0]


# speedup vs baseline: 1.1326x; 1.0669x over previous
import functools

import jax
import jax.numpy as jnp
from jax import lax
from jax.experimental import pallas as pl
from jax.experimental.pallas import tpu as pltpu
from jax.experimental.pallas import tpu_sc as plsc

F32 = jnp.float32

N_PAD = 50176
E_PAD = 819200
BA = 512
BB = 256
BC = 128
BN = 1024
BE = 8192

_MESH = plsc.VectorSubcoreMesh(core_axis_name="c", subcore_axis_name="s")


def _agg_pipeline(tbl_h, src_h, dst3_h, acc, srcv, dstv, rows,
                  gsem, ssem, ebase, nb, ba, gsz):
    ngroups = nb // gsz
    npairs = gsz // 2
    rpb = ba // 128

    def fire_gather(j, s):
        pltpu.async_copy(tbl_h.at[srcv.at[pl.ds(j * ba, ba)]],
                         rows[s], gsem[s])

    def wait_gather(s):
        pltpu.make_async_copy(tbl_h.at[srcv.at[pl.ds(0, ba)]],
                              rows[s], gsem[s]).wait()

    def fire_scatter(j, s):
        return [pltpu.async_copy(rows[s].at[pl.ds(k * 128, 128)],
                                 acc.at[dstv.at[j * rpb + k]],
                                 ssem[s], add=True)
                for k in range(rpb)]

    def group(g, carry):
        base = ebase + g * gsz * ba
        pltpu.sync_copy(src_h.at[pl.ds(base, gsz * ba)], srcv)
        row = pl.multiple_of(ebase // 128 + g * (gsz * rpb), 2)
        pltpu.sync_copy(dst3_h.at[pl.ds(row, gsz * rpb)], dstv)
        fire_gather(0, 0)
        fire_gather(1, 1)
        for p in range(npairs):
            b0 = 2 * p
            wait_gather(0)
            sc0 = fire_scatter(b0, 0)
            wait_gather(1)
            sc1 = fire_scatter(b0 + 1, 1)
            for d in sc0:
                d.wait()
            if p < npairs - 1:
                fire_gather(b0 + 2, 0)
            for d in sc1:
                d.wait()
            if p < npairs - 1:
                fire_gather(b0 + 3, 1)
        return carry

    lax.fori_loop(0, ngroups, group, 0)


def _sck_l1(xp, src, dst3, z16):
    stripe = N_PAD // 16

    @functools.partial(
        pl.kernel,
        out_type=jax.ShapeDtypeStruct((2, N_PAD, 128), F32),
        mesh=_MESH,
        compiler_params=pltpu.CompilerParams(use_tc_tiling_on_sc=False),
        scratch_types=[
            pltpu.VMEM_SHARED((N_PAD, 16), F32),
            pltpu.VMEM((10 * BA,), jnp.int32),
            pltpu.VMEM((10 * BA // 128, 128), jnp.int32),
            [pltpu.VMEM((BA, 16), F32)] * 2,
            [pltpu.SemaphoreType.DMA] * 2,
            [pltpu.SemaphoreType.DMA] * 2,
        ],
    )
    def k(xp_h, src_h, dst3_h, z_h, out_h, acc, srcv, dstv, rows,
          gsem, ssem):
        cid = lax.axis_index("c")
        sid = lax.axis_index("s")
        pltpu.sync_copy(z_h, acc.at[pl.ds(sid * stripe, stripe)])
        plsc.subcore_barrier()
        ebase = cid * (E_PAD // 2) + sid * (E_PAD // 32)
        nb = E_PAD // 32 // BA
        _agg_pipeline(xp_h, src_h, dst3_h, acc, srcv, dstv, rows,
                      gsem, ssem, ebase, nb, BA, 10)
        plsc.subcore_barrier()
        pltpu.sync_copy(acc.at[pl.ds(sid * stripe, stripe)],
                        out_h.at[cid, pl.ds(sid * stripe, stripe),
                                 pl.ds(0, 16)])

    return k(xp, src, dst3, z16)


def _sck_agg(t0, t1, t2, t3, src, dst3, z32):
    stripe = N_PAD // 16

    @functools.partial(
        pl.kernel,
        out_type=jax.ShapeDtypeStruct((N_PAD, 128), F32),
        mesh=_MESH,
        compiler_params=pltpu.CompilerParams(use_tc_tiling_on_sc=False),
        scratch_types=[
            pltpu.VMEM_SHARED((N_PAD, 32), F32),
            pltpu.VMEM((10 * BB,), jnp.int32),
            pltpu.VMEM((10 * BB // 128, 128), jnp.int32),
            [pltpu.VMEM((BB, 32), F32)] * 2,
            [pltpu.SemaphoreType.DMA] * 2,
            [pltpu.SemaphoreType.DMA] * 2,
        ],
    )
    def k(t0_h, t1_h, t2_h, t3_h, src_h, dst3_h, z_h, out_h,
          acc, srcv, dstv, rows, gsem, ssem):
        cid = lax.axis_index("c")
        sid = lax.axis_index("s")
        ebase = sid * (E_PAD // 16)
        nb = E_PAD // 16 // BB

        def do_chunk(tbl_h, c):
            pltpu.sync_copy(z_h, acc.at[pl.ds(sid * stripe, stripe)])
            plsc.subcore_barrier()
            _agg_pipeline(tbl_h, src_h, dst3_h, acc, srcv, dstv, rows,
                          gsem, ssem, ebase, nb, BB, 10)
            plsc.subcore_barrier()
            pltpu.sync_copy(acc.at[pl.ds(sid * stripe, stripe)],
                            out_h.at[pl.ds(sid * stripe, stripe),
                                     pl.ds(32 * c, 32)])

        @pl.when(cid == 0)
        def _():
            do_chunk(t0_h, 0)
            do_chunk(t1_h, 1)

        @pl.when(cid == 1)
        def _():
            do_chunk(t2_h, 2)
            do_chunk(t3_h, 3)

    return k(t0, t1, t2, t3, src, dst3, z32)


def _sck_edge_gather(a, b, src, dst):
    GSZ = 10

    @functools.partial(
        pl.kernel,
        out_type=jax.ShapeDtypeStruct((E_PAD, 128), F32),
        mesh=_MESH,
        compiler_params=pltpu.CompilerParams(use_tc_tiling_on_sc=True),
        scratch_types=[
            pltpu.VMEM((GSZ * BC,), jnp.int32),
            pltpu.VMEM((GSZ * BC,), jnp.int32),
            [pltpu.VMEM((BC, 128), F32)] * 2,
            [pltpu.VMEM((BC, 128), F32)] * 2,
            [pltpu.SemaphoreType.DMA] * 2,
            [pltpu.SemaphoreType.DMA] * 2,
            [pltpu.SemaphoreType.DMA] * 2,
        ],
    )
    def k(a_h, b_h, src_h, dst_h, g_h,
          sidx, didx, bufa, bufb, sema, semb, semw):
        cid = lax.axis_index("c")
        sid = lax.axis_index("s")
        ebase = cid * (E_PAD // 2) + sid * (E_PAD // 32)
        nb = E_PAD // 32 // BC
        ngroups = nb // GSZ
        npairs = GSZ // 2

        def fire_gathers(j, s):
            pltpu.async_copy(a_h.at[sidx.at[pl.ds(j * BC, BC)]],
                             bufa[s], sema[s])
            pltpu.async_copy(b_h.at[didx.at[pl.ds(j * BC, BC)]],
                             bufb[s], semb[s])

        def wait_gathers(s):
            pltpu.make_async_copy(a_h.at[sidx.at[pl.ds(0, BC)]],
                                  bufa[s], sema[s]).wait()
            pltpu.make_async_copy(b_h.at[didx.at[pl.ds(0, BC)]],
                                  bufb[s], semb[s]).wait()

        def add_rows(s):
            def addrow(r, carry):
                for j in range(8):
                    sl = pl.ds(j * 16, 16)
                    bufa[s][r, sl] = bufa[s][r, sl] + bufb[s][r, sl]
                return carry

            lax.fori_loop(0, BC, addrow, 0)

        def group(g, carry):
            base = ebase + g * GSZ * BC
            pltpu.sync_copy(src_h.at[pl.ds(base, GSZ * BC)], sidx)
            pltpu.sync_copy(dst_h.at[pl.ds(base, GSZ * BC)], didx)
            fire_gathers(0, 0)
            fire_gathers(1, 1)
            for p in range(npairs):
                b0 = 2 * p
                wait_gathers(0)
                add_rows(0)
                w0 = pltpu.async_copy(
                    bufa[0], g_h.at[pl.ds(base + b0 * BC, BC)], semw[0])
                wait_gathers(1)
                add_rows(1)
                w1 = pltpu.async_copy(
                    bufa[1], g_h.at[pl.ds(base + (b0 + 1) * BC, BC)],
                    semw[1])
                w0.wait()
                if p < npairs - 1:
                    fire_gathers(b0 + 2, 0)
                w1.wait()
                if p < npairs - 1:
                    fire_gathers(b0 + 3, 1)
            return carry

        lax.fori_loop(0, ngroups, group, 0)

    return k(a, b, src, dst)


def _tck1(aggx, xp, w1lp, w1rp, b1r):
    grid = (N_PAD // BN,)

    def body(aggx_r, xp_r, wl_r, wr_r, b_r, h0, h1, h2, h3, rcp_r):
        s = aggx_r[0, :, :16] + aggx_r[1, :, :16]
        r = 1.0 / jnp.maximum(s[:, 6:7], 1.0)
        y = jnp.dot(s * r, wl_r[...], preferred_element_type=F32)
        y += jnp.dot(xp_r[...], wr_r[...], preferred_element_type=F32)
        y = jnp.maximum(y + b_r[...], 0.0)
        for c, h in enumerate((h0, h1, h2, h3)):
            h[...] = y[:, 32 * c:32 * (c + 1)]
        rcp_r[...] = jnp.broadcast_to(r, (BN, 16))

    out_shape = tuple(jax.ShapeDtypeStruct((N_PAD, 32), F32)
                      for _ in range(4))
    out_shape += (jax.ShapeDtypeStruct((N_PAD, 16), F32),)
    chunk_spec = pl.BlockSpec((BN, 32), lambda i: (i, 0))
    return pl.pallas_call(
        body,
        grid=grid,
        in_specs=[
            pl.BlockSpec((2, BN, 128), lambda i: (0, i, 0)),
            pl.BlockSpec((BN, 16), lambda i: (i, 0)),
            pl.BlockSpec((16, 128), lambda i: (0, 0)),
            pl.BlockSpec((16, 128), lambda i: (0, 0)),
            pl.BlockSpec((1, 128), lambda i: (0, 0)),
        ],
        out_specs=[chunk_spec] * 4 + [pl.BlockSpec((BN, 16), lambda i: (i, 0))],
        out_shape=out_shape,
    )(aggx, xp, w1lp, w1rp, b1r)


def _tck_sage(agg, h0, h1, h2, h3, rcp, wl, wr, br):
    grid = (N_PAD // BN,)

    def body(agg_r, h0_r, h1_r, h2_r, h3_r, rcp_r, wl_r, wr_r, b_r,
             o0, o1, o2, o3):
        aggc = agg_r[...]
        h = jnp.concatenate([h0_r[...], h1_r[...], h2_r[...], h3_r[...]],
                            axis=-1)
        mean = aggc * rcp_r[:, 0:1]
        y = jnp.dot(mean, wl_r[...], preferred_element_type=F32)
        y += jnp.dot(h, wr_r[...], preferred_element_type=F32)
        y = jnp.maximum(y + b_r[...], 0.0)
        for c, o in enumerate((o0, o1, o2, o3)):
            o[...] = y[:, 32 * c:32 * (c + 1)]

    chunk_spec = pl.BlockSpec((BN, 32), lambda i: (i, 0))
    return pl.pallas_call(
        body,
        grid=grid,
        in_specs=[
            pl.BlockSpec((BN, 128), lambda i: (i, 0)),
            chunk_spec, chunk_spec, chunk_spec, chunk_spec,
            pl.BlockSpec((BN, 16), lambda i: (i, 0)),
            pl.BlockSpec((128, 128), lambda i: (0, 0)),
            pl.BlockSpec((128, 128), lambda i: (0, 0)),
            pl.BlockSpec((1, 128), lambda i: (0, 0)),
        ],
        out_specs=[chunk_spec] * 4,
        out_shape=tuple(jax.ShapeDtypeStruct((N_PAD, 32), F32)
                        for _ in range(4)),
    )(agg, h0, h1, h2, h3, rcp, wl, wr, br)


def _tck3(agg, h0, h1, h2, h3, rcp, wl, wr, br, we1l, we1r):
    grid = (N_PAD // BN,)

    def body(agg_r, h0_r, h1_r, h2_r, h3_r, rcp_r, wl_r, wr_r, b_r,
             wel_r, wer_r, a_o, b_o):
        aggc = agg_r[...]
        h = jnp.concatenate([h0_r[...], h1_r[...], h2_r[...], h3_r[...]],
                            axis=-1)
        mean = aggc * rcp_r[:, 0:1]
        y = jnp.dot(mean, wl_r[...], preferred_element_type=F32)
        y += jnp.dot(h, wr_r[...], preferred_element_type=F32)
        y = jnp.maximum(y + b_r[...], 0.0) + h
        a_o[...] = jnp.dot(y, wel_r[...], preferred_element_type=F32)
        b_o[...] = jnp.dot(y, wer_r[...], preferred_element_type=F32)

    chunk_spec = pl.BlockSpec((BN, 32), lambda i: (i, 0))
    full_spec = pl.BlockSpec((BN, 128), lambda i: (i, 0))
    w_spec = pl.BlockSpec((128, 128), lambda i: (0, 0))
    return pl.pallas_call(
        body,
        grid=grid,
        in_specs=[
            pl.BlockSpec((BN, 128), lambda i: (i, 0)),
            chunk_spec, chunk_spec, chunk_spec, chunk_spec,
            pl.BlockSpec((BN, 16), lambda i: (i, 0)),
            w_spec, w_spec,
            pl.BlockSpec((1, 128), lambda i: (0, 0)),
            w_spec, w_spec,
        ],
        out_specs=[full_spec, full_spec],
        out_shape=(jax.ShapeDtypeStruct((N_PAD, 128), F32),
                   jax.ShapeDtypeStruct((N_PAD, 128), F32)),
    )(agg, h0, h1, h2, h3, rcp, wl, wr, br, we1l, we1r)


def _tck4(g, ea, we1e, be1r, we2, be2r, w3r, be3, E):
    grid = (pl.cdiv(E, BE),)

    def body(g_r, ea_r, we1_r, b1_r, we2_r, b2_r, w3_r, b3_s, o_r):
        e1 = g_r[...]
        e1 += lax.dot_general(ea_r[...], we1_r[...],
                              (((0,), (0,)), ((), ())),
                              preferred_element_type=F32)
        e1 = jnp.maximum(e1 + b1_r[...], 0.0)
        e2 = jnp.dot(e1, we2_r[...], preferred_element_type=F32)
        e2 = jnp.maximum(e2 + b2_r[...], 0.0)
        o_r[...] = lax.dot_general(w3_r[...], e2, (((0,), (1,)), ((), ())),
                                   preferred_element_type=F32) + b3_s[0]

    return pl.pallas_call(
        body,
        grid=grid,
        in_specs=[
            pl.BlockSpec((BE, 128), lambda i: (i, 0)),
            pl.BlockSpec((11, BE), lambda i: (0, i)),
            pl.BlockSpec((11, 128), lambda i: (0, 0)),
            pl.BlockSpec((1, 128), lambda i: (0, 0)),
            pl.BlockSpec((128, 64), lambda i: (0, 0)),
            pl.BlockSpec((1, 64), lambda i: (0, 0)),
            pl.BlockSpec((64, 1), lambda i: (0, 0)),
            pl.BlockSpec(memory_space=pltpu.SMEM),
        ],
        out_specs=pl.BlockSpec((1, BE), lambda i: (0, i)),
        out_shape=jax.ShapeDtypeStruct((1, E), F32),
    )(g, ea, we1e, be1r, we2, be2r, w3r, be3)


def kernel(x, edge_index, edge_attr, W1l, W1r, b1, W2l, W2r, b2,
           W3l, W3r, b3, We1, be1, We2, be2, We3, be3):
    N = x.shape[0]
    E = edge_index.shape[1]

    xp = jnp.zeros((N_PAD, 16), F32)
    xp = xp.at[:N, :6].set(x)
    xp = xp.at[:, 6].set(1.0)

    src = edge_index[0].astype(jnp.int32)
    dst = edge_index[1].astype(jnp.int32)
    npad = E_PAD - E
    ar = lax.iota(jnp.int32, npad)
    src_p = jnp.concatenate([src, ar % N])
    dst_p = jnp.concatenate([dst, N + ar % (N_PAD - N)])
    dst3 = dst_p.reshape(E_PAD // 128, 128)

    z16 = jnp.zeros((N_PAD // 16, 16), F32)
    z32 = jnp.zeros((N_PAD // 16, 32), F32)

    pad_w = lambda w: jnp.zeros((16, 128), F32).at[:w.shape[0]].set(w)
    w1lp, w1rp = pad_w(W1l), pad_w(W1r)
    b1r, b2r, b3r = b1.reshape(1, 128), b2.reshape(1, 128), b3.reshape(1, 128)
    we1l, we1r, we1e = We1[:128], We1[128:256], We1[256:]
    be1r, be2r = be1.reshape(1, 128), be2.reshape(1, 64)
    w3r = We3

    aggx = _sck_l1(xp, src_p, dst3, z16)
    h = _tck1(aggx, xp, w1lp, w1rp, b1r)
    hc, rcp = h[:4], h[4]

    agg2 = _sck_agg(*hc, src_p, dst3, z32)
    h2c = _tck_sage(agg2, *hc, rcp, W2l, W2r, b2r)

    agg3 = _sck_agg(*h2c, src_p, dst3, z32)
    a, b = _tck3(agg3, *h2c, rcp, W3l, W3r, b3r, we1l, we1r)

    g = _sck_edge_gather(a, b, src_p, dst_p)
    return _tck4(g, edge_attr.T, we1e, be1r, We2, be2r, w3r, be3, E)[0]


# speedup vs baseline: 1.1457x; 1.0115x over previous
import functools

import jax
import jax.numpy as jnp
from jax import lax
from jax.experimental import pallas as pl
from jax.experimental.pallas import tpu as pltpu
from jax.experimental.pallas import tpu_sc as plsc

F32 = jnp.float32

N_PAD = 50176
E_PAD = 819200
BA = 512
BB = 256
BC = 128
BN = 1024
BE = 16384

_MESH = plsc.VectorSubcoreMesh(core_axis_name="c", subcore_axis_name="s")


def _agg_pipeline(tbl_h, src_h, dst3_h, acc, srcv, dstv, rows,
                  gsem, ssem, ebase, nb, ba, gsz):
    ngroups = nb // gsz
    npairs = gsz // 2
    rpb = ba // 128

    def fire_gather(j, s):
        pltpu.async_copy(tbl_h.at[srcv.at[pl.ds(j * ba, ba)]],
                         rows[s], gsem[s])

    def wait_gather(s):
        pltpu.make_async_copy(tbl_h.at[srcv.at[pl.ds(0, ba)]],
                              rows[s], gsem[s]).wait()

    def fire_scatter(j, s):
        return [pltpu.async_copy(rows[s].at[pl.ds(k * 128, 128)],
                                 acc.at[dstv.at[j * rpb + k]],
                                 ssem[s], add=True)
                for k in range(rpb)]

    def group(g, carry):
        base = ebase + g * gsz * ba
        pltpu.sync_copy(src_h.at[pl.ds(base, gsz * ba)], srcv)
        row = pl.multiple_of(ebase // 128 + g * (gsz * rpb), 2)
        pltpu.sync_copy(dst3_h.at[pl.ds(row, gsz * rpb)], dstv)
        fire_gather(0, 0)
        fire_gather(1, 1)
        for p in range(npairs):
            b0 = 2 * p
            wait_gather(0)
            sc0 = fire_scatter(b0, 0)
            wait_gather(1)
            sc1 = fire_scatter(b0 + 1, 1)
            for d in sc0:
                d.wait()
            if p < npairs - 1:
                fire_gather(b0 + 2, 0)
            for d in sc1:
                d.wait()
            if p < npairs - 1:
                fire_gather(b0 + 3, 1)
        return carry

    lax.fori_loop(0, ngroups, group, 0)


def _sck_l1(xp, src, dst3, z16):
    stripe = N_PAD // 16

    @functools.partial(
        pl.kernel,
        out_type=jax.ShapeDtypeStruct((2, N_PAD, 128), F32),
        mesh=_MESH,
        compiler_params=pltpu.CompilerParams(use_tc_tiling_on_sc=False),
        scratch_types=[
            pltpu.VMEM_SHARED((N_PAD, 16), F32),
            pltpu.VMEM((10 * BA,), jnp.int32),
            pltpu.VMEM((10 * BA // 128, 128), jnp.int32),
            [pltpu.VMEM((BA, 16), F32)] * 2,
            [pltpu.SemaphoreType.DMA] * 2,
            [pltpu.SemaphoreType.DMA] * 2,
        ],
    )
    def k(xp_h, src_h, dst3_h, z_h, out_h, acc, srcv, dstv, rows,
          gsem, ssem):
        cid = lax.axis_index("c")
        sid = lax.axis_index("s")
        pltpu.sync_copy(z_h, acc.at[pl.ds(sid * stripe, stripe)])
        plsc.subcore_barrier()
        ebase = cid * (E_PAD // 2) + sid * (E_PAD // 32)
        nb = E_PAD // 32 // BA
        _agg_pipeline(xp_h, src_h, dst3_h, acc, srcv, dstv, rows,
                      gsem, ssem, ebase, nb, BA, 10)
        plsc.subcore_barrier()
        pltpu.sync_copy(acc.at[pl.ds(sid * stripe, stripe)],
                        out_h.at[cid, pl.ds(sid * stripe, stripe),
                                 pl.ds(0, 16)])

    return k(xp, src, dst3, z16)


def _sck_agg(t0, t1, t2, t3, src, dst3, z32):
    stripe = N_PAD // 16

    @functools.partial(
        pl.kernel,
        out_type=jax.ShapeDtypeStruct((N_PAD, 128), F32),
        mesh=_MESH,
        compiler_params=pltpu.CompilerParams(use_tc_tiling_on_sc=False),
        scratch_types=[
            pltpu.VMEM_SHARED((N_PAD, 32), F32),
            pltpu.VMEM((10 * BB,), jnp.int32),
            pltpu.VMEM((10 * BB // 128, 128), jnp.int32),
            [pltpu.VMEM((BB, 32), F32)] * 2,
            [pltpu.SemaphoreType.DMA] * 2,
            [pltpu.SemaphoreType.DMA] * 2,
        ],
    )
    def k(t0_h, t1_h, t2_h, t3_h, src_h, dst3_h, z_h, out_h,
          acc, srcv, dstv, rows, gsem, ssem):
        cid = lax.axis_index("c")
        sid = lax.axis_index("s")
        ebase = sid * (E_PAD // 16)
        nb = E_PAD // 16 // BB

        def do_chunk(tbl_h, c):
            pltpu.sync_copy(z_h, acc.at[pl.ds(sid * stripe, stripe)])
            plsc.subcore_barrier()
            _agg_pipeline(tbl_h, src_h, dst3_h, acc, srcv, dstv, rows,
                          gsem, ssem, ebase, nb, BB, 10)
            plsc.subcore_barrier()
            pltpu.sync_copy(acc.at[pl.ds(sid * stripe, stripe)],
                            out_h.at[pl.ds(sid * stripe, stripe),
                                     pl.ds(32 * c, 32)])

        @pl.when(cid == 0)
        def _():
            do_chunk(t0_h, 0)
            do_chunk(t1_h, 1)

        @pl.when(cid == 1)
        def _():
            do_chunk(t2_h, 2)
            do_chunk(t3_h, 3)

    return k(t0, t1, t2, t3, src, dst3, z32)


def _sck_edge_gather(a, b, src, dst):
    GSZ = 10

    @functools.partial(
        pl.kernel,
        out_type=jax.ShapeDtypeStruct((E_PAD, 128), F32),
        mesh=_MESH,
        compiler_params=pltpu.CompilerParams(use_tc_tiling_on_sc=True),
        scratch_types=[
            pltpu.VMEM((GSZ * BC,), jnp.int32),
            pltpu.VMEM((GSZ * BC,), jnp.int32),
            [pltpu.VMEM((BC, 128), F32)] * 2,
            [pltpu.VMEM((BC, 128), F32)] * 2,
            [pltpu.SemaphoreType.DMA] * 2,
            [pltpu.SemaphoreType.DMA] * 2,
            [pltpu.SemaphoreType.DMA] * 2,
        ],
    )
    def k(a_h, b_h, src_h, dst_h, g_h,
          sidx, didx, bufa, bufb, sema, semb, semw):
        cid = lax.axis_index("c")
        sid = lax.axis_index("s")
        ebase = cid * (E_PAD // 2) + sid * (E_PAD // 32)
        nb = E_PAD // 32 // BC
        ngroups = nb // GSZ
        npairs = GSZ // 2

        def fire_gathers(j, s):
            pltpu.async_copy(a_h.at[sidx.at[pl.ds(j * BC, BC)]],
                             bufa[s], sema[s])
            pltpu.async_copy(b_h.at[didx.at[pl.ds(j * BC, BC)]],
                             bufb[s], semb[s])

        def wait_gathers(s):
            pltpu.make_async_copy(a_h.at[sidx.at[pl.ds(0, BC)]],
                                  bufa[s], sema[s]).wait()
            pltpu.make_async_copy(b_h.at[didx.at[pl.ds(0, BC)]],
                                  bufb[s], semb[s]).wait()

        def add_rows(s):
            def addrow(r, carry):
                for j in range(8):
                    sl = pl.ds(j * 16, 16)
                    bufa[s][r, sl] = bufa[s][r, sl] + bufb[s][r, sl]
                return carry

            lax.fori_loop(0, BC, addrow, 0)

        def group(g, carry):
            base = ebase + g * GSZ * BC
            pltpu.sync_copy(src_h.at[pl.ds(base, GSZ * BC)], sidx)
            pltpu.sync_copy(dst_h.at[pl.ds(base, GSZ * BC)], didx)
            fire_gathers(0, 0)
            fire_gathers(1, 1)
            for p in range(npairs):
                b0 = 2 * p
                wait_gathers(0)
                add_rows(0)
                w0 = pltpu.async_copy(
                    bufa[0], g_h.at[pl.ds(base + b0 * BC, BC)], semw[0])
                wait_gathers(1)
                add_rows(1)
                w1 = pltpu.async_copy(
                    bufa[1], g_h.at[pl.ds(base + (b0 + 1) * BC, BC)],
                    semw[1])
                w0.wait()
                if p < npairs - 1:
                    fire_gathers(b0 + 2, 0)
                w1.wait()
                if p < npairs - 1:
                    fire_gathers(b0 + 3, 1)
            return carry

        lax.fori_loop(0, ngroups, group, 0)

    return k(a, b, src, dst)


def _tck1(aggx, xp, w1lp, w1rp, b1r):
    grid = (N_PAD // BN,)

    def body(aggx_r, xp_r, wl_r, wr_r, b_r, h0, h1, h2, h3, rcp_r):
        s = aggx_r[0, :, :16] + aggx_r[1, :, :16]
        r = 1.0 / jnp.maximum(s[:, 6:7], 1.0)
        y = jnp.dot(s * r, wl_r[...], preferred_element_type=F32)
        y += jnp.dot(xp_r[...], wr_r[...], preferred_element_type=F32)
        y = jnp.maximum(y + b_r[...], 0.0)
        for c, h in enumerate((h0, h1, h2, h3)):
            h[...] = y[:, 32 * c:32 * (c + 1)]
        rcp_r[...] = jnp.broadcast_to(r, (BN, 16))

    out_shape = tuple(jax.ShapeDtypeStruct((N_PAD, 32), F32)
                      for _ in range(4))
    out_shape += (jax.ShapeDtypeStruct((N_PAD, 16), F32),)
    chunk_spec = pl.BlockSpec((BN, 32), lambda i: (i, 0))
    return pl.pallas_call(
        body,
        grid=grid,
        in_specs=[
            pl.BlockSpec((2, BN, 128), lambda i: (0, i, 0)),
            pl.BlockSpec((BN, 16), lambda i: (i, 0)),
            pl.BlockSpec((16, 128), lambda i: (0, 0)),
            pl.BlockSpec((16, 128), lambda i: (0, 0)),
            pl.BlockSpec((1, 128), lambda i: (0, 0)),
        ],
        out_specs=[chunk_spec] * 4 + [pl.BlockSpec((BN, 16), lambda i: (i, 0))],
        out_shape=out_shape,
    )(aggx, xp, w1lp, w1rp, b1r)


def _tck_sage(agg, h0, h1, h2, h3, rcp, wl, wr, br):
    grid = (N_PAD // BN,)

    def body(agg_r, h0_r, h1_r, h2_r, h3_r, rcp_r, wl_r, wr_r, b_r,
             o0, o1, o2, o3):
        aggc = agg_r[...]
        h = jnp.concatenate([h0_r[...], h1_r[...], h2_r[...], h3_r[...]],
                            axis=-1)
        mean = aggc * rcp_r[:, 0:1]
        y = jnp.dot(mean, wl_r[...], preferred_element_type=F32)
        y += jnp.dot(h, wr_r[...], preferred_element_type=F32)
        y = jnp.maximum(y + b_r[...], 0.0)
        for c, o in enumerate((o0, o1, o2, o3)):
            o[...] = y[:, 32 * c:32 * (c + 1)]

    chunk_spec = pl.BlockSpec((BN, 32), lambda i: (i, 0))
    return pl.pallas_call(
        body,
        grid=grid,
        in_specs=[
            pl.BlockSpec((BN, 128), lambda i: (i, 0)),
            chunk_spec, chunk_spec, chunk_spec, chunk_spec,
            pl.BlockSpec((BN, 16), lambda i: (i, 0)),
            pl.BlockSpec((128, 128), lambda i: (0, 0)),
            pl.BlockSpec((128, 128), lambda i: (0, 0)),
            pl.BlockSpec((1, 128), lambda i: (0, 0)),
        ],
        out_specs=[chunk_spec] * 4,
        out_shape=tuple(jax.ShapeDtypeStruct((N_PAD, 32), F32)
                        for _ in range(4)),
    )(agg, h0, h1, h2, h3, rcp, wl, wr, br)


def _tck3(agg, h0, h1, h2, h3, rcp, wl, wr, br, we1l, we1r):
    grid = (N_PAD // BN,)

    def body(agg_r, h0_r, h1_r, h2_r, h3_r, rcp_r, wl_r, wr_r, b_r,
             wel_r, wer_r, a_o, b_o):
        aggc = agg_r[...]
        h = jnp.concatenate([h0_r[...], h1_r[...], h2_r[...], h3_r[...]],
                            axis=-1)
        mean = aggc * rcp_r[:, 0:1]
        y = jnp.dot(mean, wl_r[...], preferred_element_type=F32)
        y += jnp.dot(h, wr_r[...], preferred_element_type=F32)
        y = jnp.maximum(y + b_r[...], 0.0) + h
        a_o[...] = jnp.dot(y, wel_r[...], preferred_element_type=F32)
        b_o[...] = jnp.dot(y, wer_r[...], preferred_element_type=F32)

    chunk_spec = pl.BlockSpec((BN, 32), lambda i: (i, 0))
    full_spec = pl.BlockSpec((BN, 128), lambda i: (i, 0))
    w_spec = pl.BlockSpec((128, 128), lambda i: (0, 0))
    return pl.pallas_call(
        body,
        grid=grid,
        in_specs=[
            pl.BlockSpec((BN, 128), lambda i: (i, 0)),
            chunk_spec, chunk_spec, chunk_spec, chunk_spec,
            pl.BlockSpec((BN, 16), lambda i: (i, 0)),
            w_spec, w_spec,
            pl.BlockSpec((1, 128), lambda i: (0, 0)),
            w_spec, w_spec,
        ],
        out_specs=[full_spec, full_spec],
        out_shape=(jax.ShapeDtypeStruct((N_PAD, 128), F32),
                   jax.ShapeDtypeStruct((N_PAD, 128), F32)),
    )(agg, h0, h1, h2, h3, rcp, wl, wr, br, we1l, we1r)


def _tck4(g, ea, we1e, be1r, we2, be2r, w3r, be3, E):
    grid = (pl.cdiv(E, BE),)

    def body(g_r, ea_r, we1_r, b1_r, we2_r, b2_r, w3_r, b3_s, o_r):
        e1 = g_r[...]
        e1 += lax.dot_general(ea_r[...], we1_r[...],
                              (((0,), (0,)), ((), ())),
                              preferred_element_type=F32)
        e1 = jnp.maximum(e1 + b1_r[...], 0.0)
        e2 = jnp.dot(e1, we2_r[...], preferred_element_type=F32)
        e2 = jnp.maximum(e2 + b2_r[...], 0.0)
        o_r[...] = lax.dot_general(w3_r[...], e2, (((0,), (1,)), ((), ())),
                                   preferred_element_type=F32) + b3_s[0]

    return pl.pallas_call(
        body,
        grid=grid,
        in_specs=[
            pl.BlockSpec((BE, 128), lambda i: (i, 0)),
            pl.BlockSpec((11, BE), lambda i: (0, i)),
            pl.BlockSpec((11, 128), lambda i: (0, 0)),
            pl.BlockSpec((1, 128), lambda i: (0, 0)),
            pl.BlockSpec((128, 64), lambda i: (0, 0)),
            pl.BlockSpec((1, 64), lambda i: (0, 0)),
            pl.BlockSpec((64, 1), lambda i: (0, 0)),
            pl.BlockSpec(memory_space=pltpu.SMEM),
        ],
        out_specs=pl.BlockSpec((1, BE), lambda i: (0, i)),
        out_shape=jax.ShapeDtypeStruct((1, E), F32),
    )(g, ea, we1e, be1r, we2, be2r, w3r, be3)


def kernel(x, edge_index, edge_attr, W1l, W1r, b1, W2l, W2r, b2,
           W3l, W3r, b3, We1, be1, We2, be2, We3, be3):
    N = x.shape[0]
    E = edge_index.shape[1]

    xp = jnp.zeros((N_PAD, 16), F32)
    xp = xp.at[:N, :6].set(x)
    xp = xp.at[:, 6].set(1.0)

    src = edge_index[0].astype(jnp.int32)
    dst = edge_index[1].astype(jnp.int32)
    npad = E_PAD - E
    ar = lax.iota(jnp.int32, npad)
    src_p = jnp.concatenate([src, ar % N])
    dst_p = jnp.concatenate([dst, N + ar % (N_PAD - N)])
    dst3 = dst_p.reshape(E_PAD // 128, 128)

    z16 = jnp.zeros((N_PAD // 16, 16), F32)
    z32 = jnp.zeros((N_PAD // 16, 32), F32)

    pad_w = lambda w: jnp.zeros((16, 128), F32).at[:w.shape[0]].set(w)
    w1lp, w1rp = pad_w(W1l), pad_w(W1r)
    b1r, b2r, b3r = b1.reshape(1, 128), b2.reshape(1, 128), b3.reshape(1, 128)
    we1l, we1r, we1e = We1[:128], We1[128:256], We1[256:]
    be1r, be2r = be1.reshape(1, 128), be2.reshape(1, 64)
    w3r = We3

    aggx = _sck_l1(xp, src_p, dst3, z16)
    h = _tck1(aggx, xp, w1lp, w1rp, b1r)
    hc, rcp = h[:4], h[4]

    agg2 = _sck_agg(*hc, src_p, dst3, z32)
    h2c = _tck_sage(agg2, *hc, rcp, W2l, W2r, b2r)

    agg3 = _sck_agg(*h2c, src_p, dst3, z32)
    a, b = _tck3(agg3, *h2c, rcp, W3l, W3r, b3r, we1l, we1r)

    g = _sck_edge_gather(a, b, src_p, dst_p)
    return _tck4(g, edge_attr.T, we1e, be1r, We2, be2r, w3r, be3, E)[0]
